# baseline (speedup 1.0000x reference)
import functools

import jax
import jax.numpy as jnp
from jax import lax
from jax.experimental import pallas as pl
from jax.experimental.pallas import tpu as pltpu
from jax.experimental.pallas import tpu_sc as plsc

B = 1024
L = 50
LP = 64
E = 64
V = 100000


def _make_pool():
  info = plsc.get_sparse_core_info()
  NC, NS, NL = info.num_cores, info.num_subcores, info.num_lanes
  NW = NC * NS
  assert B % NW == 0 and E == 4 * NL and LP % NL == 0
  b_per_w = B // NW

  mesh = plsc.VectorSubcoreMesh(core_axis_name="c", subcore_axis_name="s")

  GR = LP + L
  n_pairs = b_per_w // 2

  @functools.partial(
      pl.kernel,
      mesh=mesh,
      compiler_params=pltpu.CompilerParams(
          needs_layout_passes=False, use_tc_tiling_on_sc=False),
      out_type=jax.ShapeDtypeStruct((B, E), jnp.float32),
      scratch_types=[
          pltpu.VMEM((b_per_w * LP,), jnp.int32),
          pltpu.VMEM((GR, E), jnp.float32),
          pltpu.VMEM((GR, E), jnp.float32),
          pltpu.VMEM((b_per_w, E), jnp.float32),
          pltpu.SemaphoreType.DMA,
          pltpu.SemaphoreType.DMA,
      ],
  )
  def pool(idx_hbm, table_hbm, out_hbm, idx_v, buf0, buf1, out_v,
           sem0, sem1):
    wid = lax.axis_index("s") * NC + lax.axis_index("c")
    base = wid * (b_per_w * LP)
    pltpu.sync_copy(idx_hbm.at[pl.ds(base, b_per_w * LP)], idx_v)

    def gather(pair, buf, sem):
      return pltpu.make_async_copy(
          table_hbm.at[idx_v.at[pl.ds(pair * 2 * LP, GR)]],
          buf.at[pl.ds(0, GR)], sem)

    def compute(buf, off, b_local):
      a0 = jnp.zeros((NL,), jnp.float32)
      a1 = jnp.zeros((NL,), jnp.float32)
      a2 = jnp.zeros((NL,), jnp.float32)
      a3 = jnp.zeros((NL,), jnp.float32)
      for r in range(L):
        row = off + r
        v0 = buf[row, pl.ds(0, NL)]
        v1 = buf[row, pl.ds(NL, NL)]
        v2 = buf[row, pl.ds(2 * NL, NL)]
        v3 = buf[row, pl.ds(3 * NL, NL)]
        sq = (v0 * v0 + v1 * v1) + (v2 * v2 + v3 * v3)
        norm2 = jnp.broadcast_to(jnp.sum(sq), (NL,))
        ii = plsc.bitcast(norm2, jnp.int32)
        ii = jnp.int32(0x5F3759DF) - (ii >> 1)
        y = plsc.bitcast(ii, jnp.float32)
        for _ in range(2):
          y = y * (1.5 - 0.5 * norm2 * y * y)
        sr = jnp.where(norm2 > 1.0, y, 1.0)
        a0 = a0 + sr * v0
        a1 = a1 + sr * v1
        a2 = a2 + sr * v2
        a3 = a3 + sr * v3

      inv = jnp.float32(1.0 / L)
      out_v[b_local, pl.ds(0, NL)] = a0 * inv
      out_v[b_local, pl.ds(NL, NL)] = a1 * inv
      out_v[b_local, pl.ds(2 * NL, NL)] = a2 * inv
      out_v[b_local, pl.ds(3 * NL, NL)] = a3 * inv

    gather(0, buf0, sem0).start()

    def pair_body(t, carry):
      q0 = 2 * t
      gather(q0 + 1, buf1, sem1).start()
      gather(q0, buf0, sem0).wait()
      compute(buf0, 0, 2 * q0)
      compute(buf0, LP, 2 * q0 + 1)

      @pl.when(t < n_pairs // 2 - 1)
      def _():
        gather(q0 + 2, buf0, sem0).start()

      gather(q0 + 1, buf1, sem1).wait()
      compute(buf1, 0, 2 * q0 + 2)
      compute(buf1, LP, 2 * q0 + 3)
      return carry

    lax.fori_loop(0, n_pairs // 2, pair_body, 0)
    pltpu.sync_copy(out_v, out_hbm.at[pl.ds(wid * b_per_w, b_per_w)])

  return pool


_pool = _make_pool()


def _matmul_t(WT, pooled, bcol, VB=4096):
  def mk(w_ref, p_ref, b_ref, o_ref):
    mm = lax.dot_general(
        w_ref[...], p_ref[...], (((0,), (1,)), ((), ())),
        preferred_element_type=jnp.float32)
    bias = lax.dot_general(
        b_ref[...], jnp.ones((B, 1), jnp.float32), (((0,), (1,)), ((), ())),
        preferred_element_type=jnp.float32)
    o_ref[...] = mm + bias

  return pl.pallas_call(
      mk,
      grid=(pl.cdiv(V, VB),),
      in_specs=[
          pl.BlockSpec((E, VB), lambda i: (0, i)),
          pl.BlockSpec((B, E), lambda i: (0, 0)),
          pl.BlockSpec((1, VB), lambda i: (0, i)),
      ],
      out_specs=pl.BlockSpec((VB, B), lambda i: (i, 0)),
      out_shape=jax.ShapeDtypeStruct((V, B), jnp.float32),
      compiler_params=pltpu.CompilerParams(
          dimension_semantics=("parallel",)),
  )(WT, pooled, bcol)


def kernel(inputs, table, W, b):
  idx = inputs.astype(jnp.int32)
  idx_pad = jnp.zeros((B, LP), jnp.int32).at[:, :L].set(idx).reshape(-1)
  pooled = _pool(idx_pad, table)
  outT = _matmul_t(W.T, pooled, b.reshape(1, V))
  return outT.T


# speedup vs baseline: 1.5614x; 1.5614x over previous
import functools

import jax
import jax.numpy as jnp
from jax import lax
from jax.experimental import pallas as pl
from jax.experimental.pallas import tpu as pltpu
from jax.experimental.pallas import tpu_sc as plsc

B = 1024
L = 50
LP = 64
E = 64
V = 100000


def _make_pool():
  info = plsc.get_sparse_core_info()
  NC, NS, NL = info.num_cores, info.num_subcores, info.num_lanes
  NW = NC * NS
  assert B % NW == 0 and E == 4 * NL and LP % NL == 0
  b_per_w = B // NW

  mesh = plsc.VectorSubcoreMesh(core_axis_name="c", subcore_axis_name="s")

  @functools.partial(
      pl.kernel,
      mesh=mesh,
      compiler_params=pltpu.CompilerParams(
          needs_layout_passes=False, use_tc_tiling_on_sc=False),
      out_type=jax.ShapeDtypeStruct((B, E), jnp.float32),
      scratch_types=[
          pltpu.VMEM((b_per_w * LP,), jnp.int32),
          pltpu.VMEM((LP, E), jnp.float32),
          pltpu.VMEM((LP, E), jnp.float32),
          pltpu.VMEM((b_per_w, E), jnp.float32),
          pltpu.SemaphoreType.DMA,
          pltpu.SemaphoreType.DMA,
      ],
  )
  def pool(idx_hbm, table_hbm, out_hbm, idx_v, buf0, buf1, out_v,
           sem0, sem1):
    wid = lax.axis_index("s") * NC + lax.axis_index("c")
    base = wid * (b_per_w * LP)
    pltpu.sync_copy(idx_hbm.at[pl.ds(base, b_per_w * LP)], idx_v)

    def gather(b_local, buf, sem):
      return pltpu.make_async_copy(
          table_hbm.at[idx_v.at[pl.ds(b_local * LP, L)]],
          buf.at[pl.ds(0, L)], sem)

    def compute(buf, b_local):
      a0 = jnp.zeros((NL,), jnp.float32)
      a1 = jnp.zeros((NL,), jnp.float32)
      a2 = jnp.zeros((NL,), jnp.float32)
      a3 = jnp.zeros((NL,), jnp.float32)
      for row in range(L):
        v0 = buf[row, pl.ds(0, NL)]
        v1 = buf[row, pl.ds(NL, NL)]
        v2 = buf[row, pl.ds(2 * NL, NL)]
        v3 = buf[row, pl.ds(3 * NL, NL)]
        sq = (v0 * v0 + v1 * v1) + (v2 * v2 + v3 * v3)
        norm2 = jnp.broadcast_to(jnp.sum(sq), (NL,))
        ii = plsc.bitcast(norm2, jnp.int32)
        ii = jnp.int32(0x5F3759DF) - (ii >> 1)
        y = plsc.bitcast(ii, jnp.float32)
        for _ in range(2):
          y = y * (1.5 - 0.5 * norm2 * y * y)
        sr = jnp.where(norm2 > 1.0, y, 1.0)
        a0 = a0 + sr * v0
        a1 = a1 + sr * v1
        a2 = a2 + sr * v2
        a3 = a3 + sr * v3

      inv = jnp.float32(1.0 / L)
      out_v[b_local, pl.ds(0, NL)] = a0 * inv
      out_v[b_local, pl.ds(NL, NL)] = a1 * inv
      out_v[b_local, pl.ds(2 * NL, NL)] = a2 * inv
      out_v[b_local, pl.ds(3 * NL, NL)] = a3 * inv

    gather(0, buf0, sem0).start()

    def pair_body(p, carry):
      b0 = 2 * p
      gather(b0 + 1, buf1, sem1).start()
      gather(b0, buf0, sem0).wait()
      compute(buf0, b0)

      @pl.when(p < b_per_w // 2 - 1)
      def _():
        gather(b0 + 2, buf0, sem0).start()

      gather(b0 + 1, buf1, sem1).wait()
      compute(buf1, b0 + 1)
      return carry

    lax.fori_loop(0, b_per_w // 2, pair_body, 0)
    pltpu.sync_copy(out_v, out_hbm.at[pl.ds(wid * b_per_w, b_per_w)])

  return pool


_pool = _make_pool()


def _matmul_t(WT, pooled, bcol, VB=4096):
  def mk(w_ref, p_ref, b_ref, o_ref):
    mm = lax.dot_general(
        w_ref[...], p_ref[...], (((0,), (1,)), ((), ())),
        preferred_element_type=jnp.float32)
    bias = lax.dot_general(
        b_ref[...], jnp.ones((B, 1), jnp.float32), (((0,), (1,)), ((), ())),
        preferred_element_type=jnp.float32)
    o_ref[...] = mm + bias

  return pl.pallas_call(
      mk,
      grid=(pl.cdiv(V, VB),),
      in_specs=[
          pl.BlockSpec((E, VB), lambda i: (0, i)),
          pl.BlockSpec((B, E), lambda i: (0, 0)),
          pl.BlockSpec((1, VB), lambda i: (0, i)),
      ],
      out_specs=pl.BlockSpec((VB, B), lambda i: (i, 0)),
      out_shape=jax.ShapeDtypeStruct((V, B), jnp.float32),
      compiler_params=pltpu.CompilerParams(
          dimension_semantics=("parallel",)),
  )(WT, pooled, bcol)


def kernel(inputs, table, W, b):
  idx = inputs.astype(jnp.int32)
  idx_pad = jnp.zeros((B, LP), jnp.int32).at[:, :L].set(idx).reshape(-1)
  pooled = _pool(idx_pad, table)
  outT = _matmul_t(W.T, pooled, b.reshape(1, V))
  return outT.T
